# baseline (speedup 1.0000x reference)
import functools

import jax
import jax.numpy as jnp
from jax import lax
from jax.experimental import pallas as pl
from jax.experimental.pallas import tpu as pltpu
from jax.experimental.pallas import tpu_sc as plsc

N_NODES = 10000
N_EDGES = 320000
D = 128
LANES = 16

NC = 2
NS = 16
NW = NC * NS

E_PER_W = N_EDGES // NW
CHUNK = 80
NCHUNK = E_PER_W // CHUNK
NBODY = (NCHUNK - 5) // 3
R_SLICE = 624
R_TAIL_BASE = NS * R_SLICE
R_TAIL = N_NODES - R_TAIL_BASE


def _sc_aggregate(edge_flat, edge4, aff, feat):
    mesh = plsc.VectorSubcoreMesh(core_axis_name="c", subcore_axis_name="s")

    @functools.partial(
        pl.kernel,
        mesh=mesh,
        out_type=jax.ShapeDtypeStruct((NC * N_NODES, D), jnp.float32),
        scratch_types=(
            [pltpu.VMEM((CHUNK,), jnp.int32) for _ in range(3)]
            + [pltpu.VMEM((CHUNK,), jnp.float32) for _ in range(3)]
            + [pltpu.VMEM((CHUNK, D), jnp.float32) for _ in range(3)]
            + [pltpu.VMEM((NCHUNK, CHUNK), jnp.int32)]
            + [pltpu.VMEM_SHARED((N_NODES, D), jnp.float32)]
            + [pltpu.SemaphoreType.DMA for _ in range(12)]
        ),
    )
    def sc_kernel(edge_hbm, edge4_hbm, aff_hbm, feat_hbm, out_hbm,
                  s0, s1, s2, a0, a1, a2, r0, r1, r2, dst_v, agg_sh,
                  *sems):
        srcb = [s0, s1, s2]
        affb = [a0, a1, a2]
        rows = [r0, r1, r2]
        sem_s = sems[0:3]
        sem_a = sems[3:6]
        sem_g = sems[6:9]
        sem_c = sems[9:12]

        c = lax.axis_index("c")
        s = lax.axis_index("s")
        wid = s * NC + c

        def zfill(e, zcarry):
            for j in range(D // LANES):
                r0[e, pl.ds(j * LANES, LANES)] = jnp.zeros(
                    (LANES,), jnp.float32)
            return zcarry

        lax.fori_loop(0, CHUNK, zfill, 0)
        zbase = s * R_SLICE
        for k in range(7):
            pltpu.sync_copy(r0, agg_sh.at[pl.ds(zbase + k * CHUNK, CHUNK)])
        pltpu.sync_copy(r0.at[pl.ds(0, R_SLICE - 7 * CHUNK)],
                        agg_sh.at[pl.ds(zbase + 7 * CHUNK,
                                        R_SLICE - 7 * CHUNK)])

        @pl.when(s == 0)
        def _():
            pltpu.sync_copy(r0.at[pl.ds(0, R_TAIL)],
                            agg_sh.at[pl.ds(R_TAIL_BASE, R_TAIL)])

        pltpu.sync_copy(edge4_hbm.at[1, wid], dst_v)
        plsc.subcore_barrier()

        ebase = wid * E_PER_W

        def load_src(ci, k):
            return pltpu.async_copy(
                edge_hbm.at[pl.ds(ebase + ci * CHUNK, CHUNK)], srcb[k],
                sem_s[k])

        def wait_src(k):
            pltpu.make_async_copy(edge_hbm.at[pl.ds(0, CHUNK)], srcb[k],
                                  sem_s[k]).wait()

        def load_aff(ci, k):
            return pltpu.async_copy(
                aff_hbm.at[pl.ds(ebase + ci * CHUNK, CHUNK)], affb[k],
                sem_a[k])

        def wait_aff(k):
            pltpu.make_async_copy(aff_hbm.at[pl.ds(0, CHUNK)], affb[k],
                                  sem_a[k]).wait()

        def gather(k_src, k_rows):
            return pltpu.async_copy(feat_hbm.at[srcb[k_src]], rows[k_rows],
                                    sem_g[k_rows])

        def wait_gather(k):
            pltpu.make_async_copy(feat_hbm.at[pl.ds(0, CHUNK)], rows[k],
                                  sem_g[k]).wait()

        def wait_scatter(k):
            pltpu.make_async_copy(rows[k], agg_sh.at[pl.ds(0, CHUNK)],
                                  sem_c[k]).wait()

        def scale(k):
            def grp_body(g, gcarry):
                a = affb[k][pl.ds(g * LANES, LANES)]
                for l in range(LANES):
                    e = g * LANES + l
                    av = a[l]
                    for j in range(D // LANES):
                        sl = pl.ds(j * LANES, LANES)
                        rows[k][e, sl] = rows[k][e, sl] * av
                return gcarry

            lax.fori_loop(0, CHUNK // LANES, grp_body, 0)

        def step(ci, k, wait_sc=True, gath=True, pre=True):
            k2 = (k + 2) % 3
            wait_gather(k)
            if wait_sc:
                wait_scatter(k2)
            if gath:
                wait_src(k2)
                gather(k2, k2)
            wait_aff(k)
            scale(k)
            pltpu.async_copy(rows[k], agg_sh.at[dst_v.at[ci]], sem_c[k],
                             add=True)
            if pre:
                load_aff(ci + 3, k)
                load_src(ci + 3, k)

        for k in range(3):
            load_src(k, k)
            load_aff(k, k)
        wait_src(0)
        gather(0, 0)
        wait_src(1)
        gather(1, 1)

        step(0, 0, wait_sc=False)
        step(1, 1)

        def body(q, carry):
            ci = 3 * q + 2
            step(ci, 2)
            step(ci + 1, 0)
            step(ci + 2, 1)
            return carry

        lax.fori_loop(0, NBODY, body, 0)
        step(122, 2, pre=False)
        step(123, 0, gath=False, pre=False)
        step(124, 1, gath=False, pre=False)
        wait_scatter(1)

        plsc.subcore_barrier()
        rbase = s * R_SLICE
        pltpu.sync_copy(
            agg_sh.at[pl.ds(rbase, R_SLICE)],
            out_hbm.at[pl.ds(c * N_NODES + rbase, R_SLICE)])

        @pl.when(s == 0)
        def _():
            pltpu.sync_copy(
                agg_sh.at[pl.ds(R_TAIL_BASE, R_TAIL)],
                out_hbm.at[pl.ds(c * N_NODES + R_TAIL_BASE, R_TAIL)])

    return sc_kernel(edge_flat, edge4, aff, feat)


_TC_BLK = 2000
_DIMS = (((1,), (1,)), ((), ()))


def _tc_body(feat_ref, agg0_ref, agg1_ref, w_ref, b_ref, out_ref):
    w1 = w_ref[:, :D]
    w2 = w_ref[:, D:]
    acc = lax.dot_general(feat_ref[...], w1, _DIMS,
                          preferred_element_type=jnp.float32)
    agg = agg0_ref[...] + agg1_ref[...]
    acc = acc + lax.dot_general(agg, w2, _DIMS,
                                preferred_element_type=jnp.float32)
    out_ref[...] = acc + b_ref[...]


def _tc_linear(feat, partials, W, b2d):
    nblk = N_NODES // _TC_BLK
    return pl.pallas_call(
        _tc_body,
        grid=(nblk,),
        in_specs=[
            pl.BlockSpec((_TC_BLK, D), lambda i: (i, 0)),
            pl.BlockSpec((_TC_BLK, D), lambda i: (i, 0)),
            pl.BlockSpec((_TC_BLK, D), lambda i: (i + nblk, 0)),
            pl.BlockSpec((D, 2 * D), lambda i: (0, 0)),
            pl.BlockSpec((1, D), lambda i: (0, 0)),
        ],
        out_specs=pl.BlockSpec((_TC_BLK, D), lambda i: (i, 0)),
        out_shape=jax.ShapeDtypeStruct((N_NODES, D), jnp.float32),
    )(feat, partials, partials, W, b2d)


def kernel(feat, edge_index, edge_affine, W, b):
    edge_flat = edge_index.reshape(2 * N_EDGES)
    edge4 = edge_index.reshape(2, NW, NCHUNK, CHUNK)
    partials = _sc_aggregate(edge_flat, edge4, edge_affine, feat)
    return _tc_linear(feat, partials, W, b.reshape(1, D))


# speedup vs baseline: 1.1256x; 1.1256x over previous
import functools

import jax
import jax.numpy as jnp
from jax import lax
from jax.experimental import pallas as pl
from jax.experimental.pallas import tpu as pltpu
from jax.experimental.pallas import tpu_sc as plsc

N_NODES = 10000
N_EDGES = 320000
D = 128
LANES = 16

NC = 2
NS = 16
NW = NC * NS

E_PER_W = N_EDGES // NW
CHUNK = 80
NCHUNK = E_PER_W // CHUNK
NBODY = (NCHUNK - 5) // 4
R_SLICE = 624
R_TAIL_BASE = NS * R_SLICE
R_TAIL = N_NODES - R_TAIL_BASE


def _sc_aggregate(edge_flat, aff, feat):
    mesh = plsc.VectorSubcoreMesh(core_axis_name="c", subcore_axis_name="s")

    @functools.partial(
        pl.kernel,
        mesh=mesh,
        out_type=jax.ShapeDtypeStruct((NC * N_NODES, D), jnp.float32),
        scratch_types=(
            [pltpu.VMEM((CHUNK,), jnp.int32) for _ in range(4)]
            + [pltpu.VMEM((CHUNK,), jnp.float32) for _ in range(4)]
            + [pltpu.VMEM((1, CHUNK), jnp.int32) for _ in range(4)]
            + [pltpu.VMEM((CHUNK, D), jnp.float32) for _ in range(4)]
            + [pltpu.VMEM_SHARED((N_NODES, D), jnp.float32)]
            + [pltpu.SemaphoreType.DMA for _ in range(20)]
        ),
    )
    def sc_kernel(edge_hbm, aff_hbm, feat_hbm, out_hbm,
                  s0, s1, s2, s3, a0, a1, a2, a3, d0, d1, d2, d3,
                  r0, r1, r2, r3, agg_sh, *sems):
        srcb = [s0, s1, s2, s3]
        affb = [a0, a1, a2, a3]
        dstb = [d0, d1, d2, d3]
        rows = [r0, r1, r2, r3]
        sem_s = sems[0:4]
        sem_a = sems[4:8]
        sem_d = sems[8:12]
        sem_g = sems[12:16]
        sem_c = sems[16:20]

        c = lax.axis_index("c")
        s = lax.axis_index("s")
        wid = s * NC + c

        def zfill(e, zcarry):
            for j in range(D // LANES):
                r0[e, pl.ds(j * LANES, LANES)] = jnp.zeros(
                    (LANES,), jnp.float32)
            return zcarry

        lax.fori_loop(0, CHUNK, zfill, 0)
        zbase = s * R_SLICE
        for k in range(7):
            pltpu.sync_copy(r0, agg_sh.at[pl.ds(zbase + k * CHUNK, CHUNK)])
        pltpu.sync_copy(r0.at[pl.ds(0, R_SLICE - 7 * CHUNK)],
                        agg_sh.at[pl.ds(zbase + 7 * CHUNK,
                                        R_SLICE - 7 * CHUNK)])

        @pl.when(s == 0)
        def _():
            pltpu.sync_copy(r0.at[pl.ds(0, R_TAIL)],
                            agg_sh.at[pl.ds(R_TAIL_BASE, R_TAIL)])

        plsc.subcore_barrier()

        ebase = wid * E_PER_W

        def load_src(ci, k):
            return pltpu.async_copy(
                edge_hbm.at[pl.ds(ebase + ci * CHUNK, CHUNK)], srcb[k],
                sem_s[k])

        def wait_src(k):
            pltpu.make_async_copy(edge_hbm.at[pl.ds(0, CHUNK)], srcb[k],
                                  sem_s[k]).wait()

        def load_aff(ci, k):
            return pltpu.async_copy(
                aff_hbm.at[pl.ds(ebase + ci * CHUNK, CHUNK)], affb[k],
                sem_a[k])

        def wait_aff(k):
            pltpu.make_async_copy(aff_hbm.at[pl.ds(0, CHUNK)], affb[k],
                                  sem_a[k]).wait()

        def load_dst(ci, k):
            return pltpu.async_copy(
                edge_hbm.at[pl.ds(N_EDGES + ebase + ci * CHUNK, CHUNK)],
                dstb[k].at[0], sem_d[k])

        def wait_dst(k):
            pltpu.make_async_copy(edge_hbm.at[pl.ds(0, CHUNK)],
                                  dstb[k].at[0], sem_d[k]).wait()

        def gather(k_src, k_rows):
            return pltpu.async_copy(feat_hbm.at[srcb[k_src]], rows[k_rows],
                                    sem_g[k_rows])

        def wait_gather(k):
            pltpu.make_async_copy(feat_hbm.at[pl.ds(0, CHUNK)], rows[k],
                                  sem_g[k]).wait()

        def wait_scatter(k):
            pltpu.make_async_copy(rows[k], agg_sh.at[pl.ds(0, CHUNK)],
                                  sem_c[k]).wait()

        def scale(k):
            def grp_body(g, gcarry):
                a = affb[k][pl.ds(g * LANES, LANES)]
                for l in range(LANES):
                    e = g * LANES + l
                    av = a[l]
                    for j in range(D // LANES):
                        sl = pl.ds(j * LANES, LANES)
                        rows[k][e, sl] = rows[k][e, sl] * av
                return gcarry

            lax.fori_loop(0, CHUNK // LANES, grp_body, 0)

        def step(ci, k, wait_sc=True, gath=True, pre=True, dload=True,
                 clamp=False):
            kg = (k + 2) % 4
            wait_gather(k)
            if wait_sc:
                wait_scatter(kg)
            if dload:
                load_dst(ci + 2, kg)
            if gath:
                wait_src(kg)
                gather(kg, kg)
            wait_aff(k)
            scale(k)
            wait_dst(k)
            pltpu.async_copy(rows[k], agg_sh.at[dstb[k].at[0]], sem_c[k],
                             add=True)
            if pre:
                nxt = jnp.minimum(ci + 4, NCHUNK - 1) if clamp else ci + 4
                load_aff(nxt, k)
                load_src(nxt, k)

        for k in range(4):
            load_src(k, k)
            load_aff(k, k)
        load_dst(0, 0)
        load_dst(1, 1)
        wait_src(0)
        gather(0, 0)
        wait_src(1)
        gather(1, 1)

        step(0, 0, wait_sc=False)
        step(1, 1, wait_sc=False)

        def body(q, carry):
            ci = 4 * q + 2
            step(ci, 2, clamp=True)
            step(ci + 1, 3, clamp=True)
            step(ci + 2, 0, clamp=True)
            step(ci + 3, 1, clamp=True)
            return carry

        lax.fori_loop(0, NBODY, body, 0)
        step(122, 2, pre=False)
        step(123, 3, gath=False, pre=False, dload=False)
        step(124, 0, gath=False, pre=False, dload=False)
        wait_src(1)
        wait_aff(1)
        wait_scatter(3)
        wait_scatter(0)

        plsc.subcore_barrier()
        rbase = s * R_SLICE
        pltpu.sync_copy(
            agg_sh.at[pl.ds(rbase, R_SLICE)],
            out_hbm.at[pl.ds(c * N_NODES + rbase, R_SLICE)])

        @pl.when(s == 0)
        def _():
            pltpu.sync_copy(
                agg_sh.at[pl.ds(R_TAIL_BASE, R_TAIL)],
                out_hbm.at[pl.ds(c * N_NODES + R_TAIL_BASE, R_TAIL)])

    return sc_kernel(edge_flat, aff, feat)


_TC_BLK = 2000
_DIMS = (((1,), (1,)), ((), ()))


def _tc_body(feat_ref, agg0_ref, agg1_ref, w_ref, b_ref, out_ref):
    w1 = w_ref[:, :D]
    w2 = w_ref[:, D:]
    acc = lax.dot_general(feat_ref[...], w1, _DIMS,
                          preferred_element_type=jnp.float32)
    agg = agg0_ref[...] + agg1_ref[...]
    acc = acc + lax.dot_general(agg, w2, _DIMS,
                                preferred_element_type=jnp.float32)
    out_ref[...] = acc + b_ref[...]


def _tc_linear(feat, partials, W, b2d):
    nblk = N_NODES // _TC_BLK
    return pl.pallas_call(
        _tc_body,
        grid=(nblk,),
        in_specs=[
            pl.BlockSpec((_TC_BLK, D), lambda i: (i, 0)),
            pl.BlockSpec((_TC_BLK, D), lambda i: (i, 0)),
            pl.BlockSpec((_TC_BLK, D), lambda i: (i + nblk, 0)),
            pl.BlockSpec((D, 2 * D), lambda i: (0, 0)),
            pl.BlockSpec((1, D), lambda i: (0, 0)),
        ],
        out_specs=pl.BlockSpec((_TC_BLK, D), lambda i: (i, 0)),
        out_shape=jax.ShapeDtypeStruct((N_NODES, D), jnp.float32),
    )(feat, partials, partials, W, b2d)


def kernel(feat, edge_index, edge_affine, W, b):
    edge_flat = edge_index.reshape(2 * N_EDGES)
    partials = _sc_aggregate(edge_flat, edge_affine, feat)
    return _tc_linear(feat, partials, W, b.reshape(1, D))
